# baseline (speedup 1.0000x reference)
import functools

import jax
import jax.numpy as jnp
from jax import lax
from jax.experimental import pallas as pl
from jax.experimental.pallas import tpu as pltpu
from jax.experimental.pallas import tpu_sc as plsc

B = 16384
V = 1000000
D = 16
F = 16
L1 = 64
L2 = 32
R = 8
VR = V // R

NC = 2
NS = 16
NW = NC * NS
BPW = B // NW
NCHK = 2
CHL = BPW // NCHK
CHE = CHL * D


def _sc_gather_side(tab, feat, idx):
    mesh = plsc.VectorSubcoreMesh(core_axis_name="c", subcore_axis_name="s")

    @functools.partial(
        pl.kernel,
        mesh=mesh,
        compiler_params=pltpu.CompilerParams(use_tc_tiling_on_sc=False),
        out_type=[pltpu.HBM((NW, NCHK, CHE, R), jnp.float32)] * 2,
        scratch_types=[
            pltpu.VMEM((BPW,), jnp.int32),
            pltpu.VMEM((D * BPW,), jnp.int32),
            pltpu.VMEM((CHE, R), jnp.float32),
            pltpu.VMEM((CHE, R), jnp.float32),
            pltpu.SemaphoreType.DMA,
        ],
    )
    def k(tab_h, feat_h, idx_h, e_h, f_h, ix, il, e, f, gsem):
        wid = lax.axis_index("s") * NC + lax.axis_index("c")
        base = wid * BPW
        pltpu.sync_copy(idx_h.at[pl.ds(base, BPW)], ix)

        jrow = lax.iota(jnp.int32, 16) * VR

        def build(m, _):
            a = lax.shift_right_logical(ix[pl.ds(m * 16, 16)], 3)
            for l in range(16):
                sp = a.at[jnp.full((16,), l, jnp.int32)].get(
                    mode="promise_in_bounds")
                il[pl.ds(m * 256 + l * 16, 16)] = jrow + sp
            return _

        lax.fori_loop(0, BPW // 16, build, None)

        def chunk(c, _):
            sl = pl.ds(c * CHE, CHE)
            c1 = pltpu.async_copy(tab_h.at[il.at[sl]], e, gsem)
            c2 = pltpu.async_copy(feat_h.at[il.at[sl]], f, gsem)
            c1.wait()
            c2.wait()
            pltpu.sync_copy(e, e_h.at[wid, c])
            pltpu.sync_copy(f, f_h.at[wid, c])
            return _

        lax.fori_loop(0, NCHK, chunk, None)

    return k(tab, feat, idx)


BT = 2048
W128 = D * R


def _tc_body(x_ref, eu_ref, fu_ref, ev_ref, fv_ref,
             mw1, mb1, mw2, mb2, mw3, mb3,
             rw1, rb1, rw2, rb2, rw3, rb3, out_ref):
    lane = lax.broadcasted_iota(jnp.int32, (BT, W128), 1)
    giota = lax.broadcasted_iota(jnp.int32, (W128, D), 0)
    gcol = lax.broadcasted_iota(jnp.int32, (W128, D), 1)
    G = (lax.shift_right_logical(giota, 3) == gcol).astype(jnp.float32)

    def side(x_col, e_blk, f_blk, w1x, b1, w2, b2, w3, b3):
        m = lax.bitwise_and(x_col, R - 1)
        oh = (lax.bitwise_and(lane, R - 1) == m).astype(jnp.float32)
        esel = e_blk * oh
        fsel = f_blk * oh
        e16 = jnp.dot(esel, G, precision=lax.Precision.HIGHEST,
                      preferred_element_type=jnp.float32)
        h = jnp.dot(fsel, w1x[...], precision=lax.Precision.HIGHEST,
                    preferred_element_type=jnp.float32) + b1[...]
        h = jnp.maximum(h, 0.0)
        h = jnp.dot(h, w2[...], precision=lax.Precision.HIGHEST,
                    preferred_element_type=jnp.float32) + b2[...]
        h = jnp.maximum(h, 0.0)
        h = jnp.dot(h, w3[...], precision=lax.Precision.HIGHEST,
                    preferred_element_type=jnp.float32) + b3[...]
        return e16 + h

    u = side(x_ref[:, 0:1], eu_ref[...], fu_ref[...],
             mw1, mb1, mw2, mb2, mw3, mb3)
    v = side(x_ref[:, 1:2], ev_ref[...], fv_ref[...],
             rw1, rb1, rw2, rb2, rw3, rb3)
    out_ref[...] = jnp.sum(u * v, axis=1, keepdims=True)


def _tc_mlp_dot(x, eu, fu, ev, fv,
                m_w1, m_b1, m_w2, m_b2, m_w3, m_b3,
                r_w1, r_b1, r_w2, r_b2, r_w3, r_b3):
    row_spec = pl.BlockSpec((BT, W128), lambda i: (i, 0))

    def full(shape):
        return pl.BlockSpec(shape, lambda i: tuple(0 for _ in shape))

    out = pl.pallas_call(
        _tc_body,
        grid=(B // BT,),
        in_specs=[
            pl.BlockSpec((BT, 2), lambda i: (i, 0)),
            row_spec, row_spec, row_spec, row_spec,
            full((W128, L1)), full((1, L1)), full((L1, L2)), full((1, L2)),
            full((L2, D)), full((1, D)),
            full((W128, L1)), full((1, L1)), full((L1, L2)), full((1, L2)),
            full((L2, D)), full((1, D)),
        ],
        out_specs=pl.BlockSpec((BT, 1), lambda i: (i, 0)),
        out_shape=jax.ShapeDtypeStruct((B, 1), jnp.float32),
    )(x, eu, fu, ev, fv,
      jnp.repeat(m_w1, R, axis=0), m_b1.reshape(1, L1),
      m_w2, m_b2.reshape(1, L2), m_w3, m_b3.reshape(1, D),
      jnp.repeat(r_w1, R, axis=0), r_b1.reshape(1, L1),
      r_w2, r_b2.reshape(1, L2), r_w3, r_b3.reshape(1, D))
    return out.reshape(B)


def kernel(x, module_table, module_feats, m_w1, m_b1, m_w2, m_b2, m_w3, m_b3,
           runtime_table, runtime_feats, r_w1, r_b1, r_w2, r_b2, r_w3, r_b3):
    idx_u = x[:, 0]
    idx_v = x[:, 1]
    eu, fu = _sc_gather_side(
        module_table.T.reshape(-1, R), module_feats.T.reshape(-1, R), idx_u)
    ev, fv = _sc_gather_side(
        runtime_table.T.reshape(-1, R), runtime_feats.T.reshape(-1, R), idx_v)
    eu, fu, ev, fv = (a.reshape(B, W128) for a in (eu, fu, ev, fv))
    return _tc_mlp_dot(x, eu, fu, ev, fv,
                       m_w1, m_b1, m_w2, m_b2, m_w3, m_b3,
                       r_w1, r_b1, r_w2, r_b2, r_w3, r_b3)


# speedup vs baseline: 3.2726x; 3.2726x over previous
import functools

import jax
import jax.numpy as jnp
from jax import lax
from jax.experimental import pallas as pl
from jax.experimental.pallas import tpu as pltpu
from jax.experimental.pallas import tpu_sc as plsc

B = 16384
V = 1000000
D = 16
F = 16
L1 = 64
L2 = 32

NC = 2
NS = 16
NW = NC * NS
BPW = B // NW


def _sc_gather(mtab, mfeat, rtab, rfeat, idx_u, idx_v):
    mesh = plsc.VectorSubcoreMesh(core_axis_name="c", subcore_axis_name="s")

    @functools.partial(
        pl.kernel,
        mesh=mesh,
        compiler_params=pltpu.CompilerParams(use_tc_tiling_on_sc=False),
        out_type=[jax.ShapeDtypeStruct((B, D), jnp.float32)] * 4,
        scratch_types=[
            pltpu.VMEM((BPW,), jnp.int32),
            pltpu.VMEM((BPW,), jnp.int32),
            pltpu.VMEM((BPW, D), jnp.float32),
            pltpu.VMEM((BPW, D), jnp.float32),
            pltpu.VMEM((BPW, D), jnp.float32),
            pltpu.VMEM((BPW, D), jnp.float32),
            pltpu.SemaphoreType.DMA,
        ],
    )
    def k(mtab_h, mfeat_h, rtab_h, rfeat_h, iu_h, iv_h,
          eu_h, fu_h, ev_h, fv_h,
          iu, iv, eu, fu, ev, fv, sem):
        wid = lax.axis_index("s") * NC + lax.axis_index("c")
        base = wid * BPW
        pltpu.sync_copy(iu_h.at[pl.ds(base, BPW)], iu)
        pltpu.sync_copy(iv_h.at[pl.ds(base, BPW)], iv)
        c1 = pltpu.async_copy(mtab_h.at[iu], eu, sem)
        c2 = pltpu.async_copy(mfeat_h.at[iu], fu, sem)
        c3 = pltpu.async_copy(rtab_h.at[iv], ev, sem)
        c4 = pltpu.async_copy(rfeat_h.at[iv], fv, sem)
        c1.wait()
        c2.wait()
        c3.wait()
        c4.wait()
        pltpu.sync_copy(eu, eu_h.at[pl.ds(base, BPW)])
        pltpu.sync_copy(fu, fu_h.at[pl.ds(base, BPW)])
        pltpu.sync_copy(ev, ev_h.at[pl.ds(base, BPW)])
        pltpu.sync_copy(fv, fv_h.at[pl.ds(base, BPW)])

    return k(mtab, mfeat, rtab, rfeat, idx_u, idx_v)


BT = 2048


def _tc_body(eu_ref, fu_ref, ev_ref, fv_ref,
             mw1, mb1, mw2, mb2, mw3, mb3,
             rw1, rb1, rw2, rb2, rw3, rb3, out_ref):
    def mlp(f, w1, b1, w2, b2, w3, b3):
        h = jnp.dot(f, w1[...], precision=lax.Precision.HIGHEST,
                    preferred_element_type=jnp.float32) + b1[...]
        h = jnp.maximum(h, 0.0)
        h = jnp.dot(h, w2[...], precision=lax.Precision.HIGHEST,
                    preferred_element_type=jnp.float32) + b2[...]
        h = jnp.maximum(h, 0.0)
        return jnp.dot(h, w3[...], precision=lax.Precision.HIGHEST,
                       preferred_element_type=jnp.float32) + b3[...]

    u = eu_ref[...] + mlp(fu_ref[...], mw1, mb1, mw2, mb2, mw3, mb3)
    v = ev_ref[...] + mlp(fv_ref[...], rw1, rb1, rw2, rb2, rw3, rb3)
    out_ref[...] = jnp.sum(u * v, axis=1, keepdims=True)


def _tc_mlp_dot(eu, fu, ev, fv,
                m_w1, m_b1, m_w2, m_b2, m_w3, m_b3,
                r_w1, r_b1, r_w2, r_b2, r_w3, r_b3):
    row_spec = pl.BlockSpec((BT, D), lambda i: (i, 0))

    def full(shape):
        return pl.BlockSpec(shape, lambda i: tuple(0 for _ in shape))

    out = pl.pallas_call(
        _tc_body,
        grid=(B // BT,),
        in_specs=[
            row_spec, row_spec, row_spec, row_spec,
            full((F, L1)), full((1, L1)), full((L1, L2)), full((1, L2)),
            full((L2, D)), full((1, D)),
            full((F, L1)), full((1, L1)), full((L1, L2)), full((1, L2)),
            full((L2, D)), full((1, D)),
        ],
        out_specs=pl.BlockSpec((BT, 1), lambda i: (i, 0)),
        out_shape=jax.ShapeDtypeStruct((B, 1), jnp.float32),
    )(eu, fu, ev, fv,
      m_w1, m_b1.reshape(1, L1), m_w2, m_b2.reshape(1, L2),
      m_w3, m_b3.reshape(1, D),
      r_w1, r_b1.reshape(1, L1), r_w2, r_b2.reshape(1, L2),
      r_w3, r_b3.reshape(1, D))
    return out.reshape(B)


def kernel(x, module_table, module_feats, m_w1, m_b1, m_w2, m_b2, m_w3, m_b3,
           runtime_table, runtime_feats, r_w1, r_b1, r_w2, r_b2, r_w3, r_b3):
    idx_u = x[:, 0]
    idx_v = x[:, 1]
    eu, fu, ev, fv = _sc_gather(module_table, module_feats,
                                runtime_table, runtime_feats, idx_u, idx_v)
    return _tc_mlp_dot(eu, fu, ev, fv,
                       m_w1, m_b1, m_w2, m_b2, m_w3, m_b3,
                       r_w1, r_b1, r_w2, r_b2, r_w3, r_b3)
